# Initial kernel scaffold; baseline (speedup 1.0000x reference)
import jax
import jax.numpy as jnp
from jax.experimental import pallas as pl


def kernel(x, edge_index, edge_type, params):
    raise NotImplementedError("write your pallas kernel here")


# speedup vs baseline: 10.4196x; 10.4196x over previous
import functools
import jax
import jax.numpy as jnp
from jax import lax
from jax.experimental import pallas as pl
from jax.experimental.pallas import tpu as pltpu
from jax.experimental.pallas import tpu_sc as plsc

NN = 10000
NE = 320000
DD = 128
NR = 4
NL = 3
LN_EPS = 1e-5

NC = 2
NS = 16
NW = NC * NS
EPT = NE // NW
CH = 2000
NCH = EPT // CH
KK = 80
NK = CH // KK
RPT = 640
NNP = NS * RPT

NEG = -1e30
f32 = jnp.float32
i32 = jnp.int32


NB = 10
BB = NN // NB


def _dense_prep(h, relW, gateW, gateb):
    def body(h_r, relW_r, gateW_r, gateb_r, hr_r, gate_r):
        hb = h_r[...]
        for r in range(NR):
            hr_r[r] = jnp.dot(hb, relW_r[r], preferred_element_type=f32)
            gate_r[r] = jax.nn.sigmoid(
                jnp.dot(hb, gateW_r[r], preferred_element_type=f32) + gateb_r[r])

    full = lambda *dims: pl.BlockSpec(dims, lambda i: (0,) * len(dims))
    return pl.pallas_call(
        body,
        grid=(NB,),
        in_specs=[
            pl.BlockSpec((BB, DD), lambda i: (i, 0)),
            full(NR, DD, DD), full(NR, DD, DD), full(NR, 1, DD),
        ],
        out_specs=[
            pl.BlockSpec((NR, BB, DD), lambda i: (0, i, 0)),
            pl.BlockSpec((NR, BB, DD), lambda i: (0, i, 0)),
        ],
        out_shape=[
            jax.ShapeDtypeStruct((NR, NN, DD), f32),
            jax.ShapeDtypeStruct((NR, NN, DD), f32),
        ],
    )(h, relW, gateW, gateb)


def _att_proj(h, AT, atb):
    def body(h_r, AT_r, atb_r, attT_r):
        attT_r[...] = lax.dot_general(
            AT_r[...], h_r[...], (((1,), (1,)), ((), ())),
            preferred_element_type=f32) + atb_r[...]

    return pl.pallas_call(
        body,
        out_shape=jax.ShapeDtypeStruct((8, NN), f32),
    )(h, AT, atb)


def _input_proj(x, inW, inb):
    def body(x_r, w_r, b_r, h_r):
        h_r[...] = jnp.maximum(
            jnp.dot(x_r[...], w_r[...], preferred_element_type=f32) + b_r[...], 0.0)

    return pl.pallas_call(
        body,
        grid=(NB,),
        in_specs=[
            pl.BlockSpec((BB, DD), lambda i: (i, 0)),
            pl.BlockSpec((DD, DD), lambda i: (0, 0)),
            pl.BlockSpec((1, DD), lambda i: (0, 0)),
        ],
        out_specs=pl.BlockSpec((BB, DD), lambda i: (i, 0)),
        out_shape=jax.ShapeDtypeStruct((NN, DD), f32),
    )(x, inW, inb)


def _post_common(acc_r, h_r, ps_r, g_r, b_r):
    total = acc_r[0] + acc_r[1]
    sr = jnp.max(ps_r[...], axis=0, keepdims=True)
    lanes = lax.broadcasted_iota(i32, (1, DD), 1)
    cnt = jnp.sum(jnp.where((lanes < NR) & (sr > 0.0), 1.0, 0.0))
    hb = h_r[...]
    y = hb + total / jnp.maximum(cnt, 1.0)
    mu = jnp.mean(y, axis=-1, keepdims=True)
    var = jnp.mean(jnp.square(y - mu), axis=-1, keepdims=True)
    yn = (y - mu) * jax.lax.rsqrt(var + LN_EPS) * g_r[...] + b_r[...]
    return jnp.where(cnt > 0.0, jnp.maximum(yn, 0.0), hb)


def _post_mid(acc, h, ps, lng, lnb, relW, gateW, gateb):
    def body(acc_r, h_r, ps_r, g_r, b_r, relW_r, gateW_r, gateb_r,
             h2_r, hr_r, gate_r):
        hn = _post_common(acc_r, h_r, ps_r, g_r, b_r)
        h2_r[...] = hn
        for r in range(NR):
            hr_r[r] = jnp.dot(hn, relW_r[r], preferred_element_type=f32)
            gate_r[r] = jax.nn.sigmoid(
                jnp.dot(hn, gateW_r[r], preferred_element_type=f32) + gateb_r[r])

    full = lambda *dims: pl.BlockSpec(dims, lambda i: (0,) * len(dims))
    return pl.pallas_call(
        body,
        grid=(NB,),
        in_specs=[
            pl.BlockSpec((NC, BB, DD), lambda i: (0, i, 0)),
            pl.BlockSpec((BB, DD), lambda i: (i, 0)),
            full(NW, DD),
            full(1, DD), full(1, DD),
            full(NR, DD, DD), full(NR, DD, DD), full(NR, 1, DD),
        ],
        out_specs=[
            pl.BlockSpec((BB, DD), lambda i: (i, 0)),
            pl.BlockSpec((NR, BB, DD), lambda i: (0, i, 0)),
            pl.BlockSpec((NR, BB, DD), lambda i: (0, i, 0)),
        ],
        out_shape=[
            jax.ShapeDtypeStruct((NN, DD), f32),
            jax.ShapeDtypeStruct((NR, NN, DD), f32),
            jax.ShapeDtypeStruct((NR, NN, DD), f32),
        ],
    )(acc, h, ps, lng, lnb, relW, gateW, gateb)


def _post_final(acc, h, ps, lng, lnb, outW, outb):
    def body(acc_r, h_r, ps_r, g_r, b_r, w_r, ob_r, out_r):
        hn = _post_common(acc_r, h_r, ps_r, g_r, b_r)
        out_r[...] = jnp.dot(hn, w_r[...], preferred_element_type=f32) + ob_r[...]

    full = lambda *dims: pl.BlockSpec(dims, lambda i: (0,) * len(dims))
    return pl.pallas_call(
        body,
        grid=(NB,),
        in_specs=[
            pl.BlockSpec((NC, BB, DD), lambda i: (0, i, 0)),
            pl.BlockSpec((BB, DD), lambda i: (i, 0)),
            full(NW, DD),
            full(1, DD), full(1, DD),
            full(DD, DD), full(1, DD),
        ],
        out_specs=pl.BlockSpec((BB, DD), lambda i: (i, 0)),
        out_shape=jax.ShapeDtypeStruct((NN, DD), f32),
    )(acc, h, ps, lng, lnb, outW, outb)


_PASS_A_SPEC = dict(
    out_type=[
        jax.ShapeDtypeStruct((NE,), f32),
        jax.ShapeDtypeStruct((NW, DD), f32),
        jax.ShapeDtypeStruct((NW, DD), f32),
    ],
    scratch_types=[
        pltpu.VMEM((8 * NN,), f32),
        pltpu.VMEM((CH,), i32),
        pltpu.VMEM((CH,), i32),
        pltpu.VMEM((CH,), i32),
        pltpu.VMEM((CH,), f32),
        pltpu.VMEM((DD,), f32),
        pltpu.VMEM((DD,), f32),
    ],
)


def _sc_pass_a_body(attT_hbm, src_hbm, dst_hbm, typ_hbm,
                    logit_hbm, pm_hbm, ps_hbm,
                    att_v, src_v, dst_v, typ_v, log_v, pm_v, ps_v):
    cid = lax.axis_index("c")
    sid = lax.axis_index("s")
    wid = sid * NC + cid
    base = wid * EPT

    pltpu.sync_copy(attT_hbm, att_v)

    def chunk_body(c, carry):
        off = base + c * CH
        pltpu.sync_copy(src_hbm.at[pl.ds(off, CH)], src_v)
        pltpu.sync_copy(dst_hbm.at[pl.ds(off, CH)], dst_v)
        pltpu.sync_copy(typ_hbm.at[pl.ds(off, CH)], typ_v)

        def vec_body(v, carry):
            ms = list(carry[:NR])
            ss = list(carry[NR:])
            sl = pl.ds(v * 16, 16)
            s16 = src_v[sl]
            d16 = dst_v[sl]
            t16 = typ_v[sl]
            a = plsc.load_gather(att_v, [t16 * NN + s16])
            b = plsc.load_gather(att_v, [(t16 + NR) * NN + d16])
            l = a + b
            l = jnp.maximum(l, 0.2 * l)
            log_v[sl] = l
            for r in range(NR):
                mask = t16 == r
                lm = jnp.where(mask, l, NEG)
                m_new = jnp.maximum(ms[r], jnp.max(lm))
                e = jnp.where(mask, jnp.exp(l - m_new), 0.0)
                ss[r] = ss[r] * jnp.exp(ms[r] - m_new) + jnp.sum(e)
                ms[r] = m_new
            return tuple(ms) + tuple(ss)

        carry = lax.fori_loop(0, CH // 16, vec_body, carry)
        pltpu.sync_copy(log_v, logit_hbm.at[pl.ds(off, CH)])
        return carry

    init = tuple(jnp.full((16,), NEG, f32) for _ in range(NR)) \
        + tuple(jnp.zeros((16,), f32) for _ in range(NR))
    carry = lax.fori_loop(0, NCH, chunk_body, init)

    lanes = lax.broadcasted_iota(i32, (16,), 0)
    pm = jnp.full((16,), NEG, f32)
    ps = jnp.zeros((16,), f32)
    for r in range(NR):
        pm = jnp.where(lanes == r, carry[r], pm)
        ps = jnp.where(lanes == r, carry[NR + r], ps)
    pm_v[pl.ds(0, 16)] = pm
    ps_v[pl.ds(0, 16)] = ps
    for q in range(1, DD // 16):
        pm_v[pl.ds(q * 16, 16)] = jnp.full((16,), NEG, f32)
        ps_v[pl.ds(q * 16, 16)] = jnp.zeros((16,), f32)
    pltpu.sync_copy(pm_v, pm_hbm.at[wid])
    pltpu.sync_copy(ps_v, ps_hbm.at[wid])


_PASS_B_SPEC = dict(
    out_type=jax.ShapeDtypeStruct((NC, NNP, DD), f32),
    scratch_types=[
        pltpu.VMEM_SHARED((NNP, DD), f32),
        pltpu.VMEM((CH,), i32),
        pltpu.VMEM((CH,), i32),
        pltpu.VMEM((CH,), i32),
        pltpu.VMEM((CH,), f32),
        pltpu.VMEM((CH,), f32),
        pltpu.VMEM((KK,), i32),
        pltpu.VMEM((KK,), i32),
        pltpu.VMEM((KK,), i32),
        pltpu.VMEM((KK, DD), f32),
        pltpu.VMEM((KK, DD), f32),
        pltpu.VMEM((NW, DD), f32),
        pltpu.VMEM((NW, DD), f32),
        pltpu.VMEM((16,), f32),
        pltpu.VMEM((16,), f32),
        pltpu.SemaphoreType.DMA,
        pltpu.SemaphoreType.DMA,
    ],
)


def _sc_pass_b_body(hr_hbm, gate_hbm, src_hbm, dst_hbm, typ_hbm, logit_hbm,
                    pm_hbm, ps_hbm,
                    acc_hbm,
                    acc_sh, src_v, dst_v, typ_v, log_v, w_v,
                    hidx_v, gidx_v, didx_v, rows_h, rows_g,
                    pmall_v, psall_v, mtab_v, stab_v,
                    sem_h, sem_g):
    cid = lax.axis_index("c")
    sid = lax.axis_index("s")
    wid = sid * NC + cid
    base = wid * EPT

    pltpu.sync_copy(pm_hbm, pmall_v)
    pltpu.sync_copy(ps_hbm, psall_v)

    def mmax(i, m):
        return jnp.maximum(m, pmall_v[i, pl.ds(0, 16)])

    M = lax.fori_loop(0, NW, mmax, jnp.full((16,), NEG, f32))

    def msum(i, s):
        return s + psall_v[i, pl.ds(0, 16)] * jnp.exp(pmall_v[i, pl.ds(0, 16)] - M)

    S = lax.fori_loop(0, NW, msum, jnp.zeros((16,), f32))
    mtab_v[...] = M
    stab_v[...] = 1.0 / jnp.where(S > 0.0, S, 1.0)

    zero16 = jnp.zeros((16,), f32)

    def zb(i, _):
        rows_h[i // (DD // 16), pl.ds((i % (DD // 16)) * 16, 16)] = zero16
        return 0

    lax.fori_loop(0, KK * (DD // 16), zb, 0)
    for q in range(RPT // KK):
        pltpu.sync_copy(rows_h, acc_sh.at[pl.ds(sid * RPT + q * KK, KK)])
    plsc.subcore_barrier()

    def chunk_body(c, _):
        off = base + c * CH
        pltpu.sync_copy(src_hbm.at[pl.ds(off, CH)], src_v)
        pltpu.sync_copy(dst_hbm.at[pl.ds(off, CH)], dst_v)
        pltpu.sync_copy(typ_hbm.at[pl.ds(off, CH)], typ_v)
        pltpu.sync_copy(logit_hbm.at[pl.ds(off, CH)], log_v)

        def vec_body(v, _):
            sl = pl.ds(v * 16, 16)
            t16 = typ_v[sl]
            l16 = log_v[sl]
            m = plsc.load_gather(mtab_v, [t16])
            iv = plsc.load_gather(stab_v, [t16])
            w_v[sl] = jnp.exp(l16 - m) * iv
            return 0

        lax.fori_loop(0, CH // 16, vec_body, 0)

        def sub_body(k, _):
            kb = k * KK
            for v in range(KK // 16):
                sl = pl.ds(kb + v * 16, 16)
                osl = pl.ds(v * 16, 16)
                t16 = typ_v[sl]
                toff = t16 * NN
                hidx_v[osl] = src_v[sl] + toff
                d16 = dst_v[sl]
                gidx_v[osl] = d16 + toff
                didx_v[osl] = d16
            cp_h = pltpu.async_copy(hr_hbm.at[hidx_v], rows_h, sem_h)
            cp_g = pltpu.async_copy(gate_hbm.at[gidx_v], rows_g, sem_g)
            cp_h.wait()
            cp_g.wait()

            def edge_body(j, _):
                wj = plsc.load_gather(w_v, [jnp.zeros((16,), i32) + (kb + j)])
                for cc in range(DD // 16):
                    dsl = pl.ds(cc * 16, 16)
                    rows_h[j, dsl] = rows_h[j, dsl] * rows_g[j, dsl] * wj
                return 0

            lax.fori_loop(0, KK, edge_body, 0)
            pltpu.sync_copy(rows_h, acc_sh.at[didx_v], add=True)
            return 0

        lax.fori_loop(0, NK, sub_body, 0)
        return 0

    lax.fori_loop(0, NCH, chunk_body, 0)

    plsc.subcore_barrier()
    pltpu.sync_copy(acc_sh.at[pl.ds(sid * RPT, RPT)],
                    acc_hbm.at[cid, pl.ds(sid * RPT, RPT)])


@functools.lru_cache(maxsize=None)
def _sc_kernels():
    mesh = plsc.VectorSubcoreMesh(core_axis_name="c", subcore_axis_name="s",
                                  num_cores=NC, num_subcores=NS)
    cp = pltpu.CompilerParams(needs_layout_passes=False)
    pass_a = pl.kernel(_sc_pass_a_body, mesh=mesh, compiler_params=cp,
                       **_PASS_A_SPEC)
    pass_b = pl.kernel(_sc_pass_b_body, mesh=mesh, compiler_params=cp,
                       **_PASS_B_SPEC)
    return pass_a, pass_b


def kernel(x, edge_index, edge_type, params):
    src = edge_index[0]
    dst = edge_index[1]
    typ = edge_type

    relW = jnp.stack(params['rel_W'])
    gateW = jnp.stack(params['gate_W'])
    gateb = jnp.stack(params['gate_b'])[:, None, :]
    AT = jnp.concatenate(
        [jnp.stack([params['att_W'][r][:DD, 0] for r in range(NR)]),
         jnp.stack([params['att_W'][r][DD:, 0] for r in range(NR)])])
    atb = jnp.concatenate(
        [jnp.stack([params['att_b'][r] for r in range(NR)]),
         jnp.zeros((NR, 1), f32)])
    inb = params['in_b'][None, :]
    outb = params['out_b'][None, :]

    h = _input_proj(x, params['in_W'], inb)
    hr, gate = _dense_prep(h, relW, gateW, gateb)
    _sc_pass_a, _sc_pass_b = _sc_kernels()

    for layer in range(NL):
        attT = _att_proj(h, AT, atb)
        logits, pm, ps = _sc_pass_a(attT.reshape(8 * NN), src, dst, typ)
        acc = _sc_pass_b(hr.reshape(NR * NN, DD), gate.reshape(NR * NN, DD),
                         src, dst, typ, logits, pm, ps)
        lng = params['ln_g'][layer][None, :]
        lnb = params['ln_b'][layer][None, :]
        if layer < NL - 1:
            h, hr, gate = _post_mid(
                acc, h, ps, lng, lnb, relW, gateW, gateb)
        else:
            out = _post_final(acc, h, ps, lng, lnb, params['out_W'], outb)
    return out


# speedup vs baseline: 13.6007x; 1.3053x over previous
import functools
import jax
import jax.numpy as jnp
from jax import lax
from jax.experimental import pallas as pl
from jax.experimental.pallas import tpu as pltpu
from jax.experimental.pallas import tpu_sc as plsc

NN = 10000
NE = 320000
DD = 128
NR = 4
NL = 3
LN_EPS = 1e-5

NC = 2
NS = 16
NW = NC * NS
EPT = NE // NW
CH = 2000
NCH = EPT // CH
KK = 80
BCH = 400
BNK = BCH // KK
NSUB = EPT // KK
RPT = 640
NNP = NS * RPT

NEG = -1e30
f32 = jnp.float32
i32 = jnp.int32


NB = 10
BB = NN // NB


def _dense_prep(h, relW, gateW, gateb):
    def body(h_r, relW_r, gateW_r, gateb_r, hr_r, gate_r):
        hb = h_r[...]
        for r in range(NR):
            hr_r[r] = jnp.dot(hb, relW_r[r], preferred_element_type=f32)
            gate_r[r] = jax.nn.sigmoid(
                jnp.dot(hb, gateW_r[r], preferred_element_type=f32) + gateb_r[r])

    full = lambda *dims: pl.BlockSpec(dims, lambda i: (0,) * len(dims))
    return pl.pallas_call(
        body,
        grid=(NB,),
        in_specs=[
            pl.BlockSpec((BB, DD), lambda i: (i, 0)),
            full(NR, DD, DD), full(NR, DD, DD), full(NR, 1, DD),
        ],
        out_specs=[
            pl.BlockSpec((NR, BB, DD), lambda i: (0, i, 0)),
            pl.BlockSpec((NR, BB, DD), lambda i: (0, i, 0)),
        ],
        out_shape=[
            jax.ShapeDtypeStruct((NR, NN, DD), f32),
            jax.ShapeDtypeStruct((NR, NN, DD), f32),
        ],
    )(h, relW, gateW, gateb)


def _att_proj(h, AT, atb):
    def body(h_r, AT_r, atb_r, attT_r):
        attT_r[...] = lax.dot_general(
            AT_r[...], h_r[...], (((1,), (1,)), ((), ())),
            preferred_element_type=f32) + atb_r[...]

    return pl.pallas_call(
        body,
        out_shape=jax.ShapeDtypeStruct((8, NN), f32),
    )(h, AT, atb)


def _input_proj(x, inW, inb):
    def body(x_r, w_r, b_r, h_r):
        h_r[...] = jnp.maximum(
            jnp.dot(x_r[...], w_r[...], preferred_element_type=f32) + b_r[...], 0.0)

    return pl.pallas_call(
        body,
        grid=(NB,),
        in_specs=[
            pl.BlockSpec((BB, DD), lambda i: (i, 0)),
            pl.BlockSpec((DD, DD), lambda i: (0, 0)),
            pl.BlockSpec((1, DD), lambda i: (0, 0)),
        ],
        out_specs=pl.BlockSpec((BB, DD), lambda i: (i, 0)),
        out_shape=jax.ShapeDtypeStruct((NN, DD), f32),
    )(x, inW, inb)


def _post_common(acc_r, h_r, ps_r, g_r, b_r):
    total = acc_r[0] + acc_r[1]
    sr = jnp.max(ps_r[...], axis=0, keepdims=True)
    lanes = lax.broadcasted_iota(i32, (1, DD), 1)
    cnt = jnp.sum(jnp.where((lanes < NR) & (sr > 0.0), 1.0, 0.0))
    hb = h_r[...]
    y = hb + total / jnp.maximum(cnt, 1.0)
    mu = jnp.mean(y, axis=-1, keepdims=True)
    var = jnp.mean(jnp.square(y - mu), axis=-1, keepdims=True)
    yn = (y - mu) * jax.lax.rsqrt(var + LN_EPS) * g_r[...] + b_r[...]
    return jnp.where(cnt > 0.0, jnp.maximum(yn, 0.0), hb)


def _post_mid(acc, h, ps, lng, lnb, relW, gateW, gateb):
    def body(acc_r, h_r, ps_r, g_r, b_r, relW_r, gateW_r, gateb_r,
             h2_r, hr_r, gate_r):
        hn = _post_common(acc_r, h_r, ps_r, g_r, b_r)
        h2_r[...] = hn
        for r in range(NR):
            hr_r[r] = jnp.dot(hn, relW_r[r], preferred_element_type=f32)
            gate_r[r] = jax.nn.sigmoid(
                jnp.dot(hn, gateW_r[r], preferred_element_type=f32) + gateb_r[r])

    full = lambda *dims: pl.BlockSpec(dims, lambda i: (0,) * len(dims))
    return pl.pallas_call(
        body,
        grid=(NB,),
        in_specs=[
            pl.BlockSpec((NC, BB, DD), lambda i: (0, i, 0)),
            pl.BlockSpec((BB, DD), lambda i: (i, 0)),
            full(NW, DD),
            full(1, DD), full(1, DD),
            full(NR, DD, DD), full(NR, DD, DD), full(NR, 1, DD),
        ],
        out_specs=[
            pl.BlockSpec((BB, DD), lambda i: (i, 0)),
            pl.BlockSpec((NR, BB, DD), lambda i: (0, i, 0)),
            pl.BlockSpec((NR, BB, DD), lambda i: (0, i, 0)),
        ],
        out_shape=[
            jax.ShapeDtypeStruct((NN, DD), f32),
            jax.ShapeDtypeStruct((NR, NN, DD), f32),
            jax.ShapeDtypeStruct((NR, NN, DD), f32),
        ],
    )(acc, h, ps, lng, lnb, relW, gateW, gateb)


def _post_final(acc, h, ps, lng, lnb, outW, outb):
    def body(acc_r, h_r, ps_r, g_r, b_r, w_r, ob_r, out_r):
        hn = _post_common(acc_r, h_r, ps_r, g_r, b_r)
        out_r[...] = jnp.dot(hn, w_r[...], preferred_element_type=f32) + ob_r[...]

    full = lambda *dims: pl.BlockSpec(dims, lambda i: (0,) * len(dims))
    return pl.pallas_call(
        body,
        grid=(NB,),
        in_specs=[
            pl.BlockSpec((NC, BB, DD), lambda i: (0, i, 0)),
            pl.BlockSpec((BB, DD), lambda i: (i, 0)),
            full(NW, DD),
            full(1, DD), full(1, DD),
            full(DD, DD), full(1, DD),
        ],
        out_specs=pl.BlockSpec((BB, DD), lambda i: (i, 0)),
        out_shape=jax.ShapeDtypeStruct((NN, DD), f32),
    )(acc, h, ps, lng, lnb, outW, outb)


_PASS_A_SPEC = dict(
    out_type=[
        jax.ShapeDtypeStruct((NE,), f32),
        jax.ShapeDtypeStruct((NW, DD), f32),
        jax.ShapeDtypeStruct((NW, DD), f32),
    ],
    scratch_types=[
        pltpu.VMEM((8 * NN,), f32),
        pltpu.VMEM((CH,), i32),
        pltpu.VMEM((CH,), i32),
        pltpu.VMEM((CH,), i32),
        pltpu.VMEM((CH,), f32),
        pltpu.VMEM((DD,), f32),
        pltpu.VMEM((DD,), f32),
    ],
)


def _sc_pass_a_body(attT_hbm, src_hbm, dst_hbm, typ_hbm,
                    logit_hbm, pm_hbm, ps_hbm,
                    att_v, src_v, dst_v, typ_v, log_v, pm_v, ps_v):
    cid = lax.axis_index("c")
    sid = lax.axis_index("s")
    wid = sid * NC + cid
    base = wid * EPT

    pltpu.sync_copy(attT_hbm, att_v)

    def chunk_body(c, carry):
        off = base + c * CH
        pltpu.sync_copy(src_hbm.at[pl.ds(off, CH)], src_v)
        pltpu.sync_copy(dst_hbm.at[pl.ds(off, CH)], dst_v)
        pltpu.sync_copy(typ_hbm.at[pl.ds(off, CH)], typ_v)

        def vec_body(v, carry):
            ms = list(carry[:NR])
            ss = list(carry[NR:])
            sl = pl.ds(v * 16, 16)
            s16 = src_v[sl]
            d16 = dst_v[sl]
            t16 = typ_v[sl]
            a = plsc.load_gather(att_v, [t16 * NN + s16])
            b = plsc.load_gather(att_v, [(t16 + NR) * NN + d16])
            l = a + b
            l = jnp.maximum(l, 0.2 * l)
            log_v[sl] = l
            for r in range(NR):
                mask = t16 == r
                lm = jnp.where(mask, l, NEG)
                m_new = jnp.maximum(ms[r], jnp.max(lm))
                e = jnp.where(mask, jnp.exp(l - m_new), 0.0)
                ss[r] = ss[r] * jnp.exp(ms[r] - m_new) + jnp.sum(e)
                ms[r] = m_new
            return tuple(ms) + tuple(ss)

        carry = lax.fori_loop(0, CH // 16, vec_body, carry)
        pltpu.sync_copy(log_v, logit_hbm.at[pl.ds(off, CH)])
        return carry

    init = tuple(jnp.full((16,), NEG, f32) for _ in range(NR)) \
        + tuple(jnp.zeros((16,), f32) for _ in range(NR))
    carry = lax.fori_loop(0, NCH, chunk_body, init)

    lanes = lax.broadcasted_iota(i32, (16,), 0)
    pm = jnp.full((16,), NEG, f32)
    ps = jnp.zeros((16,), f32)
    for r in range(NR):
        pm = jnp.where(lanes == r, carry[r], pm)
        ps = jnp.where(lanes == r, carry[NR + r], ps)
    pm_v[pl.ds(0, 16)] = pm
    ps_v[pl.ds(0, 16)] = ps
    for q in range(1, DD // 16):
        pm_v[pl.ds(q * 16, 16)] = jnp.full((16,), NEG, f32)
        ps_v[pl.ds(q * 16, 16)] = jnp.zeros((16,), f32)
    pltpu.sync_copy(pm_v, pm_hbm.at[wid])
    pltpu.sync_copy(ps_v, ps_hbm.at[wid])


_PASS_B_SPEC = dict(
    out_type=jax.ShapeDtypeStruct((NC, NNP, DD), f32),
    scratch_types=[
        pltpu.VMEM_SHARED((NNP, DD), f32),
        pltpu.VMEM((BCH,), i32),
        pltpu.VMEM((BCH,), i32),
        pltpu.VMEM((BCH,), i32),
        pltpu.VMEM((BCH,), f32),
        pltpu.VMEM((2, BCH), f32),
        pltpu.VMEM((2, KK), i32),
        pltpu.VMEM((2, KK), i32),
        pltpu.VMEM((2, KK), i32),
        pltpu.VMEM((KK, DD), f32),
        pltpu.VMEM((KK, DD), f32),
        pltpu.VMEM((KK, DD), f32),
        pltpu.VMEM((KK, DD), f32),
        pltpu.VMEM((16,), f32),
        pltpu.VMEM((16,), f32),
        pltpu.SemaphoreType.DMA,
        pltpu.SemaphoreType.DMA,
        pltpu.SemaphoreType.DMA,
        pltpu.SemaphoreType.DMA,
    ],
)


def _sc_pass_b_body(hr_hbm, gate_hbm, src_hbm, dst_hbm, typ_hbm, logit_hbm,
                    pm_hbm, ps_hbm,
                    acc_hbm,
                    acc_sh, src_v, dst_v, typ_v, log_v, w_v,
                    hidx_v, gidx_v, didx_v, rows_ha, rows_ga, rows_hb, rows_gb,
                    mtab_v, stab_v,
                    sem_ha, sem_ga, sem_hb, sem_gb):
    cid = lax.axis_index("c")
    sid = lax.axis_index("s")
    wid = sid * NC + cid
    base = wid * EPT

    pltpu.sync_copy(pm_hbm, rows_ha.at[pl.ds(0, NW)])
    pltpu.sync_copy(ps_hbm, rows_ga.at[pl.ds(0, NW)])

    def mmax(i, m):
        return jnp.maximum(m, rows_ha[i, pl.ds(0, 16)])

    M = lax.fori_loop(0, NW, mmax, jnp.full((16,), NEG, f32))

    def msum(i, s):
        return s + rows_ga[i, pl.ds(0, 16)] * jnp.exp(rows_ha[i, pl.ds(0, 16)] - M)

    S = lax.fori_loop(0, NW, msum, jnp.zeros((16,), f32))
    mtab_v[...] = M
    stab_v[...] = 1.0 / jnp.where(S > 0.0, S, 1.0)

    zero16 = jnp.zeros((16,), f32)

    def zb(i, _):
        rows_ha[i // (DD // 16), pl.ds((i % (DD // 16)) * 16, 16)] = zero16
        return 0

    lax.fori_loop(0, KK * (DD // 16), zb, 0)
    for q in range(RPT // KK):
        pltpu.sync_copy(rows_ha, acc_sh.at[pl.ds(sid * RPT + q * KK, KK)])
    plsc.subcore_barrier()

    def stage_chunk(c):
        off = base + c * BCH
        pltpu.sync_copy(src_hbm.at[pl.ds(off, BCH)], src_v)
        pltpu.sync_copy(dst_hbm.at[pl.ds(off, BCH)], dst_v)
        pltpu.sync_copy(typ_hbm.at[pl.ds(off, BCH)], typ_v)
        pltpu.sync_copy(logit_hbm.at[pl.ds(off, BCH)], log_v)
        cp = lax.rem(c, 2)

        def vec_body(v, _):
            sl = pl.ds(v * 16, 16)
            t16 = typ_v[sl]
            l16 = log_v[sl]
            m = plsc.load_gather(mtab_v, [t16])
            iv = plsc.load_gather(stab_v, [t16])
            w_v[cp, sl] = jnp.exp(l16 - m) * iv
            return 0

        lax.fori_loop(0, BCH // 16, vec_body, 0)

    def prep(t, par):

        @pl.when(lax.rem(t, BNK) == 0)
        def _():
            stage_chunk(lax.div(t, BNK))

        kb = lax.rem(t, BNK) * KK
        for v in range(KK // 16):
            sl = pl.ds(kb + v * 16, 16)
            osl = pl.ds(v * 16, 16)
            t16 = typ_v[sl]
            toff = t16 * NN
            hidx_v[par, osl] = src_v[sl] + toff
            d16 = dst_v[sl]
            gidx_v[par, osl] = d16 + toff
            didx_v[par, osl] = d16
        if par == 0:
            pltpu.async_copy(hr_hbm.at[hidx_v.at[0]], rows_ha, sem_ha)
            pltpu.async_copy(gate_hbm.at[gidx_v.at[0]], rows_ga, sem_ga)
        else:
            pltpu.async_copy(hr_hbm.at[hidx_v.at[1]], rows_hb, sem_hb)
            pltpu.async_copy(gate_hbm.at[gidx_v.at[1]], rows_gb, sem_gb)

    def process(t, par):
        rh, rg = (rows_ha, rows_ga) if par == 0 else (rows_hb, rows_gb)
        sh, sg = (sem_ha, sem_ga) if par == 0 else (sem_hb, sem_gb)
        pltpu.make_async_copy(hr_hbm.at[hidx_v.at[par]], rh, sh).wait()
        pltpu.make_async_copy(gate_hbm.at[gidx_v.at[par]], rg, sg).wait()
        cp16 = jnp.zeros((16,), i32) + lax.rem(lax.div(t, BNK), 2)
        kb = lax.rem(t, BNK) * KK

        def edge_body(j, _):
            wj = plsc.load_gather(w_v, [cp16, jnp.zeros((16,), i32) + (kb + j)])
            for cc in range(DD // 16):
                dsl = pl.ds(cc * 16, 16)
                rh[j, dsl] = rh[j, dsl] * rg[j, dsl] * wj
            return 0

        lax.fori_loop(0, KK, edge_body, 0)
        pltpu.sync_copy(rh, acc_sh.at[didx_v.at[par]], add=True)

    prep(0, 0)

    def pair_body(i, _):
        t0 = 2 * i
        prep(t0 + 1, 1)
        process(t0, 0)
        prep(t0 + 2, 0)
        process(t0 + 1, 1)
        return 0

    lax.fori_loop(0, (NSUB - 1) // 2, pair_body, 0)
    process(NSUB - 1, 0)

    plsc.subcore_barrier()
    pltpu.sync_copy(acc_sh.at[pl.ds(sid * RPT, RPT)],
                    acc_hbm.at[cid, pl.ds(sid * RPT, RPT)])


@functools.lru_cache(maxsize=None)
def _sc_kernels():
    mesh = plsc.VectorSubcoreMesh(core_axis_name="c", subcore_axis_name="s",
                                  num_cores=NC, num_subcores=NS)
    cp = pltpu.CompilerParams(needs_layout_passes=False)
    pass_a = pl.kernel(_sc_pass_a_body, mesh=mesh, compiler_params=cp,
                       **_PASS_A_SPEC)
    pass_b = pl.kernel(_sc_pass_b_body, mesh=mesh, compiler_params=cp,
                       **_PASS_B_SPEC)
    return pass_a, pass_b


def kernel(x, edge_index, edge_type, params):
    src = edge_index[0]
    dst = edge_index[1]
    typ = edge_type

    relW = jnp.stack(params['rel_W'])
    gateW = jnp.stack(params['gate_W'])
    gateb = jnp.stack(params['gate_b'])[:, None, :]
    AT = jnp.concatenate(
        [jnp.stack([params['att_W'][r][:DD, 0] for r in range(NR)]),
         jnp.stack([params['att_W'][r][DD:, 0] for r in range(NR)])])
    atb = jnp.concatenate(
        [jnp.stack([params['att_b'][r] for r in range(NR)]),
         jnp.zeros((NR, 1), f32)])
    inb = params['in_b'][None, :]
    outb = params['out_b'][None, :]

    h = _input_proj(x, params['in_W'], inb)
    hr, gate = _dense_prep(h, relW, gateW, gateb)
    _sc_pass_a, _sc_pass_b = _sc_kernels()

    for layer in range(NL):
        attT = _att_proj(h, AT, atb)
        logits, pm, ps = _sc_pass_a(attT.reshape(8 * NN), src, dst, typ)
        acc = _sc_pass_b(hr.reshape(NR * NN, DD), gate.reshape(NR * NN, DD),
                         src, dst, typ, logits, pm, ps)
        lng = params['ln_g'][layer][None, :]
        lnb = params['ln_b'][layer][None, :]
        if layer < NL - 1:
            h, hr, gate = _post_mid(
                acc, h, ps, lng, lnb, relW, gateW, gateb)
        else:
            out = _post_final(acc, h, ps, lng, lnb, params['out_W'], outb)
    return out
